# baseline (speedup 1.0000x reference)
import functools

import jax
import jax.numpy as jnp
from jax import lax
from jax.experimental import pallas as pl
from jax.experimental.pallas import tpu as pltpu
from jax.experimental.pallas import tpu_sc as plsc

N = 10000
D = 128
R = 4
E = 320000

NC = 2
NS = 16
NW = NC * NS

EDGES = R * E
EPW = EDGES // NW
K = 80
CHUNKS = EPW // K
CB = 20
NBLK = CHUNKS // CB
NPAD = 10240
RPW = NPAD // NS

_f32 = jnp.float32

_sc_mesh = plsc.VectorSubcoreMesh(core_axis_name="c", subcore_axis_name="s")


@functools.partial(
    pl.kernel,
    out_type=[
        jax.ShapeDtypeStruct((NPAD, D), _f32),
        jax.ShapeDtypeStruct((NPAD, D), _f32),
    ],
    mesh=_sc_mesh,
    scratch_types=[
        pltpu.VMEM((CB, K), jnp.int32),
        pltpu.VMEM((CB, K), jnp.int32),
        [pltpu.VMEM((K, D), _f32)] * 4,
        pltpu.VMEM_SHARED((NPAD, D), _f32),
        [pltpu.SemaphoreType.DMA] * 4,
        [pltpu.SemaphoreType.DMA] * 4,
    ],
)
def _sc_edge_agg(table_hbm, cols_hbm, rows_hbm, zeros_hbm,
                 out0_hbm, out1_hbm, cols_v, rows_v, gb, acc, gsem, ssem):
    c = lax.axis_index("c")
    s = lax.axis_index("s")
    wid = c * NS + s

    pltpu.sync_copy(zeros_hbm.at[pl.ds(s * RPW, RPW)],
                    acc.at[pl.ds(s * RPW, RPW)])
    plsc.subcore_barrier()

    def _drain_scatter(b):
        pltpu.make_async_copy(table_hbm.at[pl.ds(0, K)], gb[b], ssem[b]).wait()

    @pl.loop(0, NBLK)
    def _(ob):
        pltpu.sync_copy(cols_hbm.at[wid, ob], cols_v)
        pltpu.sync_copy(rows_hbm.at[wid, ob], rows_v)

        for i in range(2):
            @pl.when(ob > 0)
            def _():
                _drain_scatter(i)
            pltpu.async_copy(table_hbm.at[cols_v.at[i]], gb[i], gsem[i])

        @pl.loop(0, CB // 4)
        def _(q):
            for i in range(4):
                j = 4 * q + i
                b = i
                pltpu.make_async_copy(table_hbm.at[pl.ds(0, K)],
                                      gb[b], gsem[b]).wait()
                pltpu.async_copy(gb[b], acc.at[rows_v.at[j]], ssem[b],
                                 add=True)
                nb = (i + 2) % 4

                @pl.when(j + 2 < CB)
                def _():
                    @pl.when(ob * CB + j >= 2)
                    def _():
                        _drain_scatter(nb)
                    pltpu.async_copy(table_hbm.at[cols_v.at[j + 2]],
                                     gb[nb], gsem[nb])

    for i in range(4):
        _drain_scatter(i)

    plsc.subcore_barrier()

    @pl.when(c == 0)
    def _():
        pltpu.sync_copy(acc.at[pl.ds(s * RPW, RPW)],
                        out0_hbm.at[pl.ds(s * RPW, RPW)])

    @pl.when(c == 1)
    def _():
        pltpu.sync_copy(acc.at[pl.ds(s * RPW, RPW)],
                        out1_hbm.at[pl.ds(s * RPW, RPW)])


_BM = 400


def _mm_body(x_ref, w_ref, o_ref):
    o_ref[...] = lax.dot_general(x_ref[...], w_ref[0],
                                 dimension_numbers=(((1,), (1,)), ((), ())),
                                 preferred_element_type=_f32)


def _combine_body(y_ref, p0_ref, p1_ref, b_ref, o_ref):
    o_ref[...] = y_ref[...] + p0_ref[...] + p1_ref[...] + b_ref[...]


def kernel(x, adjs, W_rels, W_self, b_self):
    W_stack = jnp.concatenate([W_self[None], W_rels], axis=0)

    table = pl.pallas_call(
        _mm_body,
        grid=(R + 1, N // _BM),
        in_specs=[
            pl.BlockSpec((_BM, D), lambda r, i: (i, 0)),
            pl.BlockSpec((1, D, D), lambda r, i: (r, 0, 0)),
        ],
        out_specs=pl.BlockSpec((_BM, D), lambda r, i: (r * (N // _BM) + i, 0)),
        out_shape=jax.ShapeDtypeStruct(((R + 1) * N, D), _f32),
    )(x, W_stack)

    cols_w = (adjs[:, 1, :]
              + (N * (jnp.arange(R, dtype=jnp.int32) + 1))[:, None]
              ).reshape(NW, NBLK, CB, K)
    rows_w = adjs[:, 0, :].reshape(NW, NBLK, CB, K)
    zeros = jnp.zeros((NPAD, D), _f32)

    p0, p1 = _sc_edge_agg(table, cols_w, rows_w, zeros)

    out = pl.pallas_call(
        _combine_body,
        grid=(N // _BM,),
        in_specs=[
            pl.BlockSpec((_BM, D), lambda i: (i, 0)),
            pl.BlockSpec((_BM, D), lambda i: (i, 0)),
            pl.BlockSpec((_BM, D), lambda i: (i, 0)),
            pl.BlockSpec((1, D), lambda i: (0, 0)),
        ],
        out_specs=pl.BlockSpec((_BM, D), lambda i: (i, 0)),
        out_shape=jax.ShapeDtypeStruct((N, D), _f32),
    )(table, p0, p1, b_self.reshape(1, D))
    return out


# speedup vs baseline: 1.1671x; 1.1671x over previous
import functools

import jax
import jax.numpy as jnp
from jax import lax
from jax.experimental import pallas as pl
from jax.experimental.pallas import tpu as pltpu
from jax.experimental.pallas import tpu_sc as plsc

N = 10000
D = 128
R = 4
E = 320000

NC = 2
NS = 16
NW = NC * NS

EDGES = R * E
EPW = EDGES // NW
K = 80
CHUNKS = EPW // K
CB = 20
NBLK = CHUNKS // CB
NPAD = 10240
RPW = NPAD // NS

_f32 = jnp.float32

_sc_mesh = plsc.VectorSubcoreMesh(core_axis_name="c", subcore_axis_name="s")


@functools.partial(
    pl.kernel,
    out_type=[
        jax.ShapeDtypeStruct((NPAD, D), _f32),
        jax.ShapeDtypeStruct((NPAD, D), _f32),
    ],
    mesh=_sc_mesh,
    scratch_types=[
        pltpu.VMEM((CB, K), jnp.int32),
        pltpu.VMEM((CB, K), jnp.int32),
        [pltpu.VMEM((K, D), _f32)] * 4,
        pltpu.VMEM_SHARED((NPAD, D), _f32),
        [pltpu.SemaphoreType.DMA] * 4,
        [pltpu.SemaphoreType.DMA] * 4,
    ],
)
def _sc_edge_agg(table_hbm, idx_hbm, zeros_hbm,
                 out0_hbm, out1_hbm, cols_v, rows_v, gb, acc, gsem, ssem):
    c = lax.axis_index("c")
    s = lax.axis_index("s")
    wid = c * NS + s
    rel = wid // (NW // R)
    wsub = wid % (NW // R)

    pltpu.sync_copy(zeros_hbm.at[pl.ds(s * RPW, RPW)],
                    acc.at[pl.ds(s * RPW, RPW)])
    plsc.subcore_barrier()

    def _drain_scatter(b):
        pltpu.make_async_copy(table_hbm.at[pl.ds(0, K)], gb[b], ssem[b]).wait()

    @pl.loop(0, NBLK)
    def _(ob):
        pltpu.sync_copy(idx_hbm.at[rel, 1, wsub, ob], cols_v)
        pltpu.sync_copy(idx_hbm.at[rel, 0, wsub, ob], rows_v)

        for i in range(2):
            @pl.when(ob > 0)
            def _():
                _drain_scatter(i)
            pltpu.async_copy(table_hbm.at[cols_v.at[i]], gb[i], gsem[i])

        @pl.loop(0, CB // 4)
        def _(q):
            for i in range(4):
                j = 4 * q + i
                b = i
                pltpu.make_async_copy(table_hbm.at[pl.ds(0, K)],
                                      gb[b], gsem[b]).wait()
                pltpu.async_copy(gb[b], acc.at[rows_v.at[j]], ssem[b],
                                 add=True)
                nb = (i + 2) % 4

                @pl.when(j + 2 < CB)
                def _():
                    @pl.when(ob * CB + j >= 2)
                    def _():
                        _drain_scatter(nb)
                    pltpu.async_copy(table_hbm.at[cols_v.at[j + 2]],
                                     gb[nb], gsem[nb])

    for i in range(4):
        _drain_scatter(i)

    plsc.subcore_barrier()

    @pl.when(c == 0)
    def _():
        pltpu.sync_copy(acc.at[pl.ds(s * RPW, RPW)],
                        out0_hbm.at[pl.ds(s * RPW, RPW)])

    @pl.when(c == 1)
    def _():
        pltpu.sync_copy(acc.at[pl.ds(s * RPW, RPW)],
                        out1_hbm.at[pl.ds(s * RPW, RPW)])


_BM = 400


def _mm_body(x_ref, w_ref, o_ref):
    o_ref[...] = lax.dot_general(x_ref[...], w_ref[0],
                                 dimension_numbers=(((1,), (1,)), ((), ())),
                                 preferred_element_type=_f32)


def _combine_body(y_ref, p0_ref, p1_ref, b_ref, o_ref):
    o_ref[...] = y_ref[...] + p0_ref[...] + p1_ref[...] + b_ref[...]


def kernel(x, adjs, W_rels, W_self, b_self):
    W_stack = jnp.concatenate([W_self[None], W_rels], axis=0)

    table = pl.pallas_call(
        _mm_body,
        grid=(R + 1,),
        in_specs=[
            pl.BlockSpec((N, D), lambda r: (0, 0)),
            pl.BlockSpec((1, D, D), lambda r: (r, 0, 0)),
        ],
        out_specs=pl.BlockSpec((N, D), lambda r: (r, 0)),
        out_shape=jax.ShapeDtypeStruct(((R + 1) * N, D), _f32),
    )(x, W_stack)

    offs = jnp.stack(
        [jnp.zeros((R,), jnp.int32),
         N * (jnp.arange(R, dtype=jnp.int32) + 1)], axis=1)
    idx_w = (adjs.reshape(R, 2, NW // R, NBLK, CB, K)
             + offs.reshape(R, 2, 1, 1, 1, 1))
    zeros = jnp.zeros((NPAD, D), _f32)

    p0, p1 = _sc_edge_agg(table, idx_w, zeros)

    out = pl.pallas_call(
        _combine_body,
        grid=(N // _BM,),
        in_specs=[
            pl.BlockSpec((_BM, D), lambda i: (i, 0)),
            pl.BlockSpec((_BM, D), lambda i: (i, 0)),
            pl.BlockSpec((_BM, D), lambda i: (i, 0)),
            pl.BlockSpec((1, D), lambda i: (0, 0)),
        ],
        out_specs=pl.BlockSpec((_BM, D), lambda i: (i, 0)),
        out_shape=jax.ShapeDtypeStruct((N, D), _f32),
    )(table, p0, p1, b_self.reshape(1, D))
    return out


# speedup vs baseline: 1.2452x; 1.0669x over previous
import functools

import jax
import jax.numpy as jnp
from jax import lax
from jax.experimental import pallas as pl
from jax.experimental.pallas import tpu as pltpu
from jax.experimental.pallas import tpu_sc as plsc

N = 10000
D = 128
R = 4
E = 320000

NC = 2
NS = 16
NW = NC * NS

EDGES = R * E
EPW = EDGES // NW
K = 80
CHUNKS = EPW // K
CB = 20
NBLK = CHUNKS // CB
RP0 = 624
RPL = N - (NS - 1) * RP0

_f32 = jnp.float32

_sc_mesh = plsc.VectorSubcoreMesh(core_axis_name="c", subcore_axis_name="s")


@functools.partial(
    pl.kernel,
    out_type=[
        jax.ShapeDtypeStruct((N, D), _f32),
        jax.ShapeDtypeStruct((N, D), _f32),
    ],
    mesh=_sc_mesh,
    scratch_types=[
        pltpu.VMEM((2 * CB * K,), jnp.int32),
        pltpu.VMEM((2 * CB, K), jnp.int32),
        [pltpu.VMEM((K, D), _f32)] * 4,
        pltpu.VMEM_SHARED((N, D), _f32),
        [pltpu.SemaphoreType.DMA] * 4,
        [pltpu.SemaphoreType.DMA] * 4,
        pltpu.SemaphoreType.DMA,
    ],
)
def _sc_edge_agg(table_hbm, cols_hbm, rows_hbm, zeros_hbm,
                 out0_hbm, out1_hbm, cols_v, rows_v, gb, acc, gsem, ssem,
                 isem):
    c = lax.axis_index("c")
    s = lax.axis_index("s")
    wid = c * NS + s
    rel = wid // (NW // R)
    wsub = wid % (NW // R)
    e_off = wsub * EPW
    CBK = CB * K

    @pl.when(s < NS - 1)
    def _():
        pltpu.sync_copy(zeros_hbm.at[pl.ds(s * RP0, RP0)],
                        acc.at[pl.ds(s * RP0, RP0)])

    @pl.when(s == NS - 1)
    def _():
        pltpu.sync_copy(zeros_hbm.at[pl.ds((NS - 1) * RP0, RPL)],
                        acc.at[pl.ds((NS - 1) * RP0, RPL)])

    g_off = rel * E + e_off
    pltpu.sync_copy(cols_hbm.at[pl.ds(g_off, CBK)],
                    cols_v.at[pl.ds(0, CBK)])
    pltpu.sync_copy(rows_hbm.at[rel, wsub, 0], rows_v.at[pl.ds(0, CB)])
    plsc.subcore_barrier()

    def _drain_scatter(b):
        pltpu.make_async_copy(table_hbm.at[pl.ds(0, K)], gb[b], ssem[b]).wait()

    @pl.loop(0, NBLK)
    def _(ob):
        par = lax.rem(ob, 2)
        base = par * CBK
        nbase = (1 - par) * CBK

        @pl.when(ob + 1 < NBLK)
        def _():
            pltpu.async_copy(cols_hbm.at[pl.ds(g_off + (ob + 1) * CBK, CBK)],
                             cols_v.at[pl.ds(nbase, CBK)], isem)
            pltpu.async_copy(rows_hbm.at[rel, wsub, ob + 1],
                             rows_v.at[pl.ds((1 - par) * CB, CB)], isem)

        for i in range(2):
            @pl.when(ob > 0)
            def _():
                _drain_scatter(i)
            pltpu.async_copy(
                table_hbm.at[cols_v.at[pl.ds(base + i * K, K)]],
                gb[i], gsem[i])

        @pl.loop(0, CB // 4)
        def _(q):
            for i in range(4):
                j = 4 * q + i
                b = i
                pltpu.make_async_copy(table_hbm.at[pl.ds(0, K)],
                                      gb[b], gsem[b]).wait()
                pltpu.async_copy(
                    gb[b], acc.at[rows_v.at[par * CB + j]],
                    ssem[b], add=True)
                nb = (i + 2) % 4

                @pl.when(j + 2 < CB)
                def _():
                    @pl.when(ob * CB + j >= 2)
                    def _():
                        _drain_scatter(nb)
                    pltpu.async_copy(
                        table_hbm.at[cols_v.at[pl.ds(base + (j + 2) * K, K)]],
                        gb[nb], gsem[nb])

        @pl.when(ob + 1 < NBLK)
        def _():
            pltpu.make_async_copy(cols_hbm.at[pl.ds(g_off, CBK)],
                                  cols_v.at[pl.ds(nbase, CBK)], isem).wait()
            pltpu.make_async_copy(rows_hbm.at[rel, wsub, 0],
                                  rows_v.at[pl.ds((1 - par) * CB, CB)],
                                  isem).wait()

    for i in range(4):
        _drain_scatter(i)

    plsc.subcore_barrier()

    def _writeout(out_hbm):
        @pl.when(s < NS - 1)
        def _():
            pltpu.sync_copy(acc.at[pl.ds(s * RP0, RP0)],
                            out_hbm.at[pl.ds(s * RP0, RP0)])

        @pl.when(s == NS - 1)
        def _():
            pltpu.sync_copy(acc.at[pl.ds((NS - 1) * RP0, RPL)],
                            out_hbm.at[pl.ds((NS - 1) * RP0, RPL)])

    @pl.when(c == 0)
    def _():
        _writeout(out0_hbm)

    @pl.when(c == 1)
    def _():
        _writeout(out1_hbm)


_BM = 400


def _mm_body(x_ref, w_ref, o_ref):
    o_ref[...] = lax.dot_general(x_ref[...], w_ref[0],
                                 dimension_numbers=(((1,), (1,)), ((), ())),
                                 preferred_element_type=_f32)


def _combine_body(y_ref, p0_ref, p1_ref, b_ref, o_ref):
    o_ref[...] = y_ref[...] + p0_ref[...] + p1_ref[...] + b_ref[...]


def kernel(x, adjs, W_rels, W_self, b_self):
    W_stack = jnp.concatenate([W_self[None], W_rels], axis=0)

    table = pl.pallas_call(
        _mm_body,
        grid=(R + 1,),
        in_specs=[
            pl.BlockSpec((N, D), lambda r: (0, 0)),
            pl.BlockSpec((1, D, D), lambda r: (r, 0, 0)),
        ],
        out_specs=pl.BlockSpec((N, D), lambda r: (r, 0)),
        out_shape=jax.ShapeDtypeStruct(((R + 1) * N, D), _f32),
    )(x, W_stack)

    cols_flat = (adjs[:, 1, :]
                 + (N * (jnp.arange(R, dtype=jnp.int32) + 1))[:, None]
                 ).reshape(R * E)
    rows_w = adjs[:, 0, :].reshape(R, NW // R, NBLK, CB, K)
    zeros = jnp.zeros((N, D), _f32)

    p0, p1 = _sc_edge_agg(table, cols_flat, rows_w, zeros)

    out = pl.pallas_call(
        _combine_body,
        grid=(N // _BM,),
        in_specs=[
            pl.BlockSpec((_BM, D), lambda i: (i, 0)),
            pl.BlockSpec((_BM, D), lambda i: (i, 0)),
            pl.BlockSpec((_BM, D), lambda i: (i, 0)),
            pl.BlockSpec((1, D), lambda i: (0, 0)),
        ],
        out_specs=pl.BlockSpec((_BM, D), lambda i: (i, 0)),
        out_shape=jax.ShapeDtypeStruct((N, D), _f32),
    )(table, p0, p1, b_self.reshape(1, D))
    return out
